# baseline (speedup 1.0000x reference)
import functools

import jax
import jax.numpy as jnp
from jax import lax
from jax.experimental import pallas as pl
from jax.experimental.pallas import tpu as pltpu
from jax.experimental.pallas import tpu_sc as plsc

Z_NUM = 8192
Z_DIM = 64
N_TOK = 8192
ROW_T = 1024
ARG_C = 2048


def _combined_body(z_ref, zn_ref, en_ref, emb_ref,
                   idx_ref, counts_ref, se_ref, shd_ref):
    pid = pl.program_id(0)
    nphase = N_TOK // ROW_T

    @pl.when(pid < nphase)
    def _argmin_phase():
        _argmin_work(z_ref, zn_ref, en_ref, emb_ref, idx_ref, counts_ref, pid)

    @pl.when(pid >= nphase)
    def _lse_phase():
        _lse_work(emb_ref, se_ref, shd_ref, pid - nphase)


def _argmin_work(z_ref, zn_ref, en_ref, emb_ref, idx_ref, counts_ref, pid):
    z = z_ref[...]
    e = emb_ref[...]
    z2b = (-2.0 * z).astype(jnp.bfloat16)
    eb = e.astype(jnp.bfloat16)
    zn = zn_ref[...][:, :1]
    en = en_ref[...]

    acc_v = jnp.full((ROW_T, 1), jnp.inf, jnp.float32)
    acc_i = jnp.zeros((ROW_T, 1), jnp.int32)
    for k in range(Z_NUM // ARG_C):
        s2k = lax.dot_general(z2b, eb[k * ARG_C:(k + 1) * ARG_C, :],
                              (((1,), (1,)), ((), ())),
                              preferred_element_type=jnp.float32)
        dk = (zn + en[:, k * ARG_C:(k + 1) * ARG_C]) + s2k
        mk = jnp.min(dk, axis=1, keepdims=True)
        colsk = lax.broadcasted_iota(jnp.int32, dk.shape, 1) + k * ARG_C
        ik = jnp.min(jnp.where(dk == mk, colsk, jnp.int32(Z_NUM)),
                     axis=1, keepdims=True)
        take = mk < acc_v
        acc_i = jnp.where(take, ik, acc_i)
        acc_v = jnp.where(take, mk.astype(jnp.bfloat16).astype(jnp.float32),
                          acc_v)
    idx = acc_i[:, 0]
    idx_ref[0, 0, :] = idx

    cols = lax.broadcasted_iota(jnp.int32, (ROW_T, Z_NUM), 1)
    onehot = (idx[:, None] == cols).astype(jnp.bfloat16)
    ones8 = jnp.ones((8, ROW_T), jnp.bfloat16)
    part = lax.dot_general(ones8, onehot, (((1,), (0,)), ((), ())),
                           preferred_element_type=jnp.float32)[:1, :]

    @pl.when(pid == 0)
    def _():
        counts_ref[...] = jnp.zeros_like(counts_ref)

    counts_ref[...] += part


def _lse_work(emb_ref, se_ref, shd_ref, pid):
    er = emb_ref[pl.ds(pid * ROW_T, ROW_T), :]
    e = emb_ref[...]
    g = lax.dot_general(er.astype(jnp.bfloat16), e.astype(jnp.bfloat16),
                        (((1,), (1,)), ((), ())),
                        preferred_element_type=jnp.float32)
    m = jnp.max(g, axis=1, keepdims=True)
    sh = g - m
    se_ref[0, 0, :] = jnp.sum(jnp.exp(sh), axis=1)
    erb = er.astype(jnp.bfloat16)
    gq = lax.dot_general(erb, erb, (((1,), (1,)), ((), ())),
                         preferred_element_type=jnp.float32)
    rows = lax.broadcasted_iota(jnp.int32, (ROW_T, ROW_T), 0)
    cols = lax.broadcasted_iota(jnp.int32, (ROW_T, ROW_T), 1)
    gd = jnp.sum(jnp.where(cols == rows, gq, 0.0), axis=1)
    shd_ref[0, 0, :] = gd - m[:, 0]


def _final_body(z_ref, zvq_ref, counts_ref, qut_ref, enc_ref, perp_ref):
    diff = zvq_ref[...] - z_ref[...]
    loss = jnp.sum(diff * diff)
    qut_ref[...] = jnp.reshape(loss, (1, 1))
    enc_ref[...] = jnp.reshape(loss, (1, 1))
    p = counts_ref[...] * (1.0 / N_TOK)
    ent = -jnp.sum(p * jnp.log(p + 1e-10))
    perp_ref[...] = jnp.reshape(jnp.exp(ent), (1, 1))


def _sc_gather(embedding, idx):
    lanes = 128
    table = jnp.pad(embedding, ((0, 0), (0, lanes - Z_DIM)))
    info = plsc.get_sparse_core_info()
    nw = info.num_cores * info.num_subcores
    b_per_w = N_TOK // nw
    mesh = plsc.VectorSubcoreMesh(core_axis_name="c", subcore_axis_name="s")

    @functools.partial(
        pl.kernel, mesh=mesh,
        out_type=jax.ShapeDtypeStruct((N_TOK, lanes), jnp.float32),
        scratch_types=[
            pltpu.VMEM((b_per_w,), jnp.int32),
            pltpu.VMEM((b_per_w, lanes), jnp.float32),
            pltpu.SemaphoreType.DMA,
        ],
    )
    def gather(table_hbm, idx_hbm, out_hbm, idx_v, rows_v, sem):
        wid = lax.axis_index("s") * info.num_cores + lax.axis_index("c")
        base = wid * b_per_w
        pltpu.sync_copy(idx_hbm.at[pl.ds(base, b_per_w)], idx_v)
        pltpu.async_copy(table_hbm.at[idx_v], rows_v, sem).wait()
        pltpu.sync_copy(rows_v, out_hbm.at[pl.ds(base, b_per_w)])

    return gather(table, idx)[:, :Z_DIM]


def kernel(z, embedding):
    b, d, t = z.shape
    z_flat = jnp.transpose(z, (0, 2, 1)).reshape(-1, d)
    zn = jnp.sum(z_flat ** 2, axis=1)
    en = jnp.sum(embedding ** 2, axis=1)
    grid = N_TOK // ROW_T
    egrid = Z_NUM // ROW_T
    zn_b = jnp.broadcast_to(zn[:, None], (N_TOK, 8))
    en_r = en.reshape(1, Z_NUM)

    idx3, counts, se3, shd3 = pl.pallas_call(
        _combined_body,
        grid=(grid + egrid,),
        in_specs=[
            pl.BlockSpec((ROW_T, Z_DIM), lambda i: (jnp.minimum(i, 7), 0)),
            pl.BlockSpec((ROW_T, 8), lambda i: (jnp.minimum(i, 7), 0)),
            pl.BlockSpec((1, Z_NUM), lambda i: (0, 0)),
            pl.BlockSpec((Z_NUM, Z_DIM), lambda i: (0, 0)),
        ],
        out_specs=[
            pl.BlockSpec((1, 1, ROW_T), lambda i: (jnp.minimum(i, 7), 0, 0)),
            pl.BlockSpec((1, Z_NUM), lambda i: (0, 0)),
            pl.BlockSpec((1, 1, ROW_T),
                         lambda i: (jnp.maximum(i - 8, 0), 0, 0)),
            pl.BlockSpec((1, 1, ROW_T),
                         lambda i: (jnp.maximum(i - 8, 0), 0, 0)),
        ],
        out_shape=[
            jax.ShapeDtypeStruct((grid, 1, ROW_T), jnp.int32),
            jax.ShapeDtypeStruct((1, Z_NUM), jnp.float32),
            jax.ShapeDtypeStruct((egrid, 1, ROW_T), jnp.float32),
            jax.ShapeDtypeStruct((egrid, 1, ROW_T), jnp.float32),
        ],
    )(z_flat, zn_b, en_r, embedding)
    idx = idx3.reshape(N_TOK)
    logp_diag = shd3.reshape(Z_NUM) - jnp.log(se3.reshape(Z_NUM))
    sparsity = -jnp.mean(logp_diag)

    z_vq = _sc_gather(embedding, idx)

    qut, enc, perp = pl.pallas_call(
        _final_body,
        in_specs=[
            pl.BlockSpec((N_TOK, Z_DIM), lambda: (0, 0)),
            pl.BlockSpec((N_TOK, Z_DIM), lambda: (0, 0)),
            pl.BlockSpec((1, Z_NUM), lambda: (0, 0)),
        ],
        out_specs=[pl.BlockSpec((1, 1), lambda: (0, 0))] * 3,
        out_shape=[jax.ShapeDtypeStruct((1, 1), jnp.float32)] * 3,
    )(z_flat, z_vq, counts)

    out0 = jnp.transpose(z_vq.reshape(b, t, d), (0, 2, 1))
    return (out0, qut[0, 0], enc[0, 0], perp[0, 0], sparsity)


# speedup vs baseline: 1.0211x; 1.0211x over previous
import functools

import jax
import jax.numpy as jnp
from jax import lax
from jax.experimental import pallas as pl
from jax.experimental.pallas import tpu as pltpu
from jax.experimental.pallas import tpu_sc as plsc

Z_NUM = 8192
Z_DIM = 64
N_TOK = 8192
ROW_T = 1024
ARG_C = 2048


def _argmin_body(z_ref, zn_ref, en_ref, emb_ref, idx_ref, counts_ref):
    pid = pl.program_id(0)
    z = z_ref[...]
    e = emb_ref[...]
    z2b = (-2.0 * z).astype(jnp.bfloat16)
    eb = e.astype(jnp.bfloat16)
    zn = zn_ref[...][:, :1]
    en = en_ref[...]

    acc_v = jnp.full((ROW_T, 1), jnp.inf, jnp.float32)
    acc_i = jnp.zeros((ROW_T, 1), jnp.int32)
    for k in range(Z_NUM // ARG_C):
        s2k = lax.dot_general(z2b, eb[k * ARG_C:(k + 1) * ARG_C, :],
                              (((1,), (1,)), ((), ())),
                              preferred_element_type=jnp.float32)
        dk = (zn + en[:, k * ARG_C:(k + 1) * ARG_C]) + s2k
        mk = jnp.min(dk, axis=1, keepdims=True)
        colsk = lax.broadcasted_iota(jnp.int32, dk.shape, 1) + k * ARG_C
        ik = jnp.min(jnp.where(dk == mk, colsk, jnp.int32(Z_NUM)),
                     axis=1, keepdims=True)
        take = mk < acc_v
        acc_i = jnp.where(take, ik, acc_i)
        acc_v = jnp.where(take, mk.astype(jnp.bfloat16).astype(jnp.float32),
                          acc_v)
    idx = acc_i[:, 0]
    idx_ref[0, 0, :] = idx

    cols = lax.broadcasted_iota(jnp.int32, (ROW_T, Z_NUM), 1)
    onehot = (idx[:, None] == cols).astype(jnp.bfloat16)
    ones8 = jnp.ones((8, ROW_T), jnp.bfloat16)
    part = lax.dot_general(ones8, onehot, (((1,), (0,)), ((), ())),
                           preferred_element_type=jnp.float32)[:1, :]

    @pl.when(pid == 0)
    def _():
        counts_ref[...] = jnp.zeros_like(counts_ref)

    counts_ref[...] += part


def _lse_body(erow_ref, emb_ref, se_ref, shd_ref):
    pid = pl.program_id(0)
    er = erow_ref[...]
    e = emb_ref[...]
    g = lax.dot_general(er.astype(jnp.bfloat16), e.astype(jnp.bfloat16),
                        (((1,), (1,)), ((), ())),
                        preferred_element_type=jnp.float32)
    m = jnp.max(g, axis=1, keepdims=True)
    sh = g - m
    se_ref[0, 0, :] = jnp.sum(jnp.exp(sh), axis=1)
    erb = er.astype(jnp.bfloat16)
    gq = lax.dot_general(erb, erb, (((1,), (1,)), ((), ())),
                         preferred_element_type=jnp.float32)
    rows = lax.broadcasted_iota(jnp.int32, (ROW_T, ROW_T), 0)
    cols = lax.broadcasted_iota(jnp.int32, (ROW_T, ROW_T), 1)
    gd = jnp.sum(jnp.where(cols == rows, gq, 0.0), axis=1)
    shd_ref[0, 0, :] = gd - m[:, 0]


def _final_body(z_ref, zvq_ref, counts_ref, qut_ref, enc_ref, perp_ref):
    diff = zvq_ref[...] - z_ref[...]
    loss = jnp.sum(diff * diff)
    qut_ref[...] = jnp.reshape(loss, (1, 1))
    enc_ref[...] = jnp.reshape(loss, (1, 1))
    p = counts_ref[...] * (1.0 / N_TOK)
    ent = -jnp.sum(p * jnp.log(p + 1e-10))
    perp_ref[...] = jnp.reshape(jnp.exp(ent), (1, 1))


def _sc_gather(embedding, idx):
    lanes = 128
    table = jnp.pad(embedding, ((0, 0), (0, lanes - Z_DIM)))
    info = plsc.get_sparse_core_info()
    nw = info.num_cores * info.num_subcores
    b_per_w = N_TOK // nw
    mesh = plsc.VectorSubcoreMesh(core_axis_name="c", subcore_axis_name="s")

    @functools.partial(
        pl.kernel, mesh=mesh,
        out_type=jax.ShapeDtypeStruct((N_TOK, lanes), jnp.float32),
        scratch_types=[
            pltpu.VMEM((b_per_w,), jnp.int32),
            pltpu.VMEM((b_per_w, lanes), jnp.float32),
            pltpu.SemaphoreType.DMA,
        ],
    )
    def gather(table_hbm, idx_hbm, out_hbm, idx_v, rows_v, sem):
        wid = lax.axis_index("s") * info.num_cores + lax.axis_index("c")
        base = wid * b_per_w
        pltpu.sync_copy(idx_hbm.at[pl.ds(base, b_per_w)], idx_v)
        pltpu.async_copy(table_hbm.at[idx_v], rows_v, sem).wait()
        pltpu.sync_copy(rows_v, out_hbm.at[pl.ds(base, b_per_w)])

    return gather(table, idx)[:, :Z_DIM]


def kernel(z, embedding):
    b, d, t = z.shape
    z_flat = jnp.transpose(z, (0, 2, 1)).reshape(-1, d)
    zn = jnp.sum(z_flat ** 2, axis=1)
    en = jnp.sum(embedding ** 2, axis=1)
    grid = N_TOK // ROW_T
    zn_b = jnp.broadcast_to(zn[:, None], (N_TOK, 128))
    en_r = en.reshape(1, Z_NUM)

    idx3, counts = pl.pallas_call(
        _argmin_body,
        grid=(grid,),
        in_specs=[
            pl.BlockSpec((ROW_T, Z_DIM), lambda i: (i, 0)),
            pl.BlockSpec((ROW_T, 128), lambda i: (i, 0)),
            pl.BlockSpec((1, Z_NUM), lambda i: (0, 0)),
            pl.BlockSpec((Z_NUM, Z_DIM), lambda i: (0, 0)),
        ],
        out_specs=[
            pl.BlockSpec((1, 1, ROW_T), lambda i: (i, 0, 0)),
            pl.BlockSpec((1, Z_NUM), lambda i: (0, 0)),
        ],
        out_shape=[
            jax.ShapeDtypeStruct((grid, 1, ROW_T), jnp.int32),
            jax.ShapeDtypeStruct((1, Z_NUM), jnp.float32),
        ],
    )(z_flat, zn_b, en_r, embedding)
    idx = idx3.reshape(N_TOK)

    egrid = Z_NUM // ROW_T
    se3, shd3 = pl.pallas_call(
        _lse_body,
        grid=(egrid,),
        in_specs=[
            pl.BlockSpec((ROW_T, Z_DIM), lambda i: (i, 0)),
            pl.BlockSpec((Z_NUM, Z_DIM), lambda i: (0, 0)),
        ],
        out_specs=[
            pl.BlockSpec((1, 1, ROW_T), lambda i: (i, 0, 0)),
            pl.BlockSpec((1, 1, ROW_T), lambda i: (i, 0, 0)),
        ],
        out_shape=[
            jax.ShapeDtypeStruct((egrid, 1, ROW_T), jnp.float32),
            jax.ShapeDtypeStruct((egrid, 1, ROW_T), jnp.float32),
        ],
    )(embedding, embedding)
    logp_diag = shd3.reshape(Z_NUM) - jnp.log(se3.reshape(Z_NUM))
    sparsity = -jnp.mean(logp_diag)

    z_vq = _sc_gather(embedding, idx)

    qut, enc, perp = pl.pallas_call(
        _final_body,
        in_specs=[
            pl.BlockSpec((N_TOK, Z_DIM), lambda: (0, 0)),
            pl.BlockSpec((N_TOK, Z_DIM), lambda: (0, 0)),
            pl.BlockSpec((1, Z_NUM), lambda: (0, 0)),
        ],
        out_specs=[pl.BlockSpec((1, 1), lambda: (0, 0))] * 3,
        out_shape=[jax.ShapeDtypeStruct((1, 1), jnp.float32)] * 3,
    )(z_flat, z_vq, counts)

    out0 = jnp.transpose(z_vq.reshape(b, t, d), (0, 2, 1))
    return (out0, qut[0, 0], enc[0, 0], perp[0, 0], sparsity)
